# baseline (speedup 1.0000x reference)
import functools

import jax
import jax.numpy as jnp
from jax import lax
from jax.experimental import pallas as pl
from jax.experimental.pallas import tpu as pltpu
from jax.experimental.pallas import tpu_sc as plsc

N_INPUT = 1000
LAYER = 1000
N_LAYERS = 10
N_NODES = N_LAYERS * LAYER
IN_DEG = 16
N_COMPUTED = N_LAYERS - 1

LANES = 16
NUM_TILES = 16
NODES_PER_TILE = 64
GROUPS = NODES_PER_TILE // LANES
LAST_BASE = LAYER - NODES_PER_TILE
SLAB = NODES_PER_TILE * IN_DEG
XCHG = 1024


def _body(src_hbm, w_hbm, in_hbm, out_hbm,
          out_buf, my_src, my_w, pub, shared, sem0, sem1):
    c = lax.axis_index("c")
    t = lax.axis_index("s")

    @pl.when(c == 0)
    def _run():
        base = jnp.where(t == NUM_TILES - 1, LAST_BASE, t * NODES_PER_TILE)
        ebase = base * IN_DEG
        early = [
            pltpu.async_copy(src_hbm.at[pl.ds(ebase, SLAB)],
                             my_src.at[pl.ds(0, SLAB)], sem0),
            pltpu.async_copy(w_hbm.at[pl.ds(ebase, SLAB)],
                             my_w.at[pl.ds(0, SLAB)], sem0),
            pltpu.async_copy(in_hbm, out_buf.at[pl.ds(0, N_INPUT)], sem0),
        ]
        late = []
        for l in range(1, N_COMPUTED):
            off = ebase + l * LAYER * IN_DEG
            late.append(pltpu.async_copy(
                src_hbm.at[pl.ds(off, SLAB)],
                my_src.at[pl.ds(l * SLAB, SLAB)], sem1))
            late.append(pltpu.async_copy(
                w_hbm.at[pl.ds(off, SLAB)],
                my_w.at[pl.ds(l * SLAB, SLAB)], sem1))
        with jax.named_scope("stage_wait"):
            for cp in early:
                cp.wait()

        lane16 = jax.lax.iota(jnp.int32, 16) * IN_DEG

        def compute_layer(l):
            cbases = [lane16 + (l * SLAB + g * (LANES * IN_DEG))
                      for g in range(GROUPS)]

            def gstep(j, accs):
                out = []
                for g in range(GROUPS):
                    idx = cbases[g] + j
                    sv = plsc.load_gather(my_src, [idx])
                    wv = plsc.load_gather(my_w, [idx])
                    gv = plsc.load_gather(out_buf, [sv])
                    out.append(accs[g] + wv * gv)
                return tuple(out)

            zero = jnp.zeros((LANES,), jnp.float32)
            accs = plsc.parallel_loop(
                0, IN_DEG, unroll=2, carry=(zero,) * GROUPS
            )(gstep)
            for g in range(GROUPS):
                y = 1.0 / (1.0 + jnp.exp(-accs[g]))
                pub[pl.ds(g * LANES, LANES)] = y

        def exchange(l):
            sel = (l & 1) * XCHG
            pltpu.sync_copy(
                pub,
                shared.at[pl.ds(pl.multiple_of(sel + base, 8), NODES_PER_TILE)])
            plsc.subcore_barrier()
            pltpu.sync_copy(
                shared.at[pl.ds(pl.multiple_of(sel, 8), LAYER)],
                out_buf.at[pl.ds(pl.multiple_of((l + 1) * LAYER, 8), LAYER)],
            )

        with jax.named_scope("layer0"):
            compute_layer(0)
            with jax.named_scope("late_wait"):
                for cp in late:
                    cp.wait()
            exchange(0)

        with jax.named_scope("mid_layers"):
            def lstep(l, _):
                compute_layer(l)
                exchange(l)
                return 0
            lax.fori_loop(1, N_COMPUTED - 1, lstep, 0)

        with jax.named_scope("last_layer"):
            compute_layer(N_COMPUTED - 1)
            pltpu.sync_copy(pub, out_hbm.at[pl.ds(base, NODES_PER_TILE)])


@jax.jit
def _run_net(src_all, w_all, inputs):
    mesh = plsc.VectorSubcoreMesh(
        core_axis_name="c", subcore_axis_name="s", num_cores=1
    )
    f = functools.partial(
        pl.kernel,
        mesh=mesh,
        compiler_params=pltpu.CompilerParams(needs_layout_passes=False),
        out_type=jax.ShapeDtypeStruct((LAYER,), jnp.float32),
        scratch_types=[
            pltpu.VMEM((N_NODES,), jnp.float32),
            pltpu.VMEM((N_COMPUTED * SLAB,), jnp.int32),
            pltpu.VMEM((N_COMPUTED * SLAB,), jnp.float32),
            pltpu.VMEM((NODES_PER_TILE,), jnp.float32),
            pltpu.VMEM_SHARED((2 * XCHG,), jnp.float32),
            pltpu.SemaphoreType.DMA,
            pltpu.SemaphoreType.DMA,
        ],
    )(_body)
    return f(src_all, w_all, inputs)


def kernel(inputs, edge_weight, edge_src, edge_dst):
    del edge_dst
    return _run_net(edge_src, edge_weight, inputs)


# speedup vs baseline: 1.0376x; 1.0376x over previous
import functools

import jax
import jax.numpy as jnp
from jax import lax
from jax.experimental import pallas as pl
from jax.experimental.pallas import tpu as pltpu
from jax.experimental.pallas import tpu_sc as plsc

N_INPUT = 1000
LAYER = 1000
N_LAYERS = 10
N_NODES = N_LAYERS * LAYER
IN_DEG = 16
N_COMPUTED = N_LAYERS - 1

LANES = 16
NUM_TILES = 16
NODES_PER_TILE = 64
GROUPS = NODES_PER_TILE // LANES
LAST_BASE = LAYER - NODES_PER_TILE
SLAB = NODES_PER_TILE * IN_DEG
XCHG = 1024


def _body(src_hbm, w_hbm, in_hbm, out_hbm,
          out_buf, my_src, my_w, pub, shared, sem0, sem1):
    c = lax.axis_index("c")
    t = lax.axis_index("s")

    @pl.when(c == 0)
    def _run():
        base = jnp.where(t == NUM_TILES - 1, LAST_BASE, t * NODES_PER_TILE)
        ebase = base * IN_DEG
        early = [
            pltpu.async_copy(src_hbm.at[pl.ds(ebase, SLAB)],
                             my_src.at[pl.ds(0, SLAB)], sem0),
            pltpu.async_copy(w_hbm.at[pl.ds(ebase, SLAB)],
                             my_w.at[pl.ds(0, SLAB)], sem0),
            pltpu.async_copy(in_hbm, out_buf.at[pl.ds(0, N_INPUT)], sem0),
        ]
        late = []
        for l in range(1, N_COMPUTED):
            off = ebase + l * LAYER * IN_DEG
            late.append(pltpu.async_copy(
                src_hbm.at[pl.ds(off, SLAB)],
                my_src.at[pl.ds(l * SLAB, SLAB)], sem1))
            late.append(pltpu.async_copy(
                w_hbm.at[pl.ds(off, SLAB)],
                my_w.at[pl.ds(l * SLAB, SLAB)], sem1))
        with jax.named_scope("stage_wait"):
            for cp in early:
                cp.wait()

        lane16 = jax.lax.iota(jnp.int32, 16) * IN_DEG

        def compute_layer(l):
            for g in range(GROUPS):
                cbase = lane16 + (l * SLAB + g * (LANES * IN_DEG))

                def gstep(j, acc, cbase=cbase):
                    idx = cbase + j
                    sv = plsc.load_gather(my_src, [idx])
                    wv = plsc.load_gather(my_w, [idx])
                    gv = plsc.load_gather(out_buf, [sv])
                    return acc + wv * gv

                acc = plsc.parallel_loop(
                    0, IN_DEG, unroll=16, carry=jnp.zeros((LANES,), jnp.float32)
                )(gstep)
                y = 1.0 / (1.0 + jnp.exp(-acc))
                pub[pl.ds(g * LANES, LANES)] = y

        def exchange(l):
            sel = (l & 1) * XCHG
            pltpu.sync_copy(
                pub,
                shared.at[pl.ds(pl.multiple_of(sel + base, 8), NODES_PER_TILE)])
            plsc.subcore_barrier()
            pltpu.sync_copy(
                shared.at[pl.ds(pl.multiple_of(sel, 8), LAYER)],
                out_buf.at[pl.ds(pl.multiple_of((l + 1) * LAYER, 8), LAYER)],
            )

        with jax.named_scope("layer0"):
            compute_layer(0)
            with jax.named_scope("late_wait"):
                for cp in late:
                    cp.wait()
            exchange(0)

        with jax.named_scope("mid_layers"):
            def lstep(l, _):
                compute_layer(l)
                exchange(l)
                return 0
            lax.fori_loop(1, N_COMPUTED - 1, lstep, 0)

        with jax.named_scope("last_layer"):
            compute_layer(N_COMPUTED - 1)
            pltpu.sync_copy(pub, out_hbm.at[pl.ds(base, NODES_PER_TILE)])


@jax.jit
def _run_net(src_all, w_all, inputs):
    mesh = plsc.VectorSubcoreMesh(
        core_axis_name="c", subcore_axis_name="s", num_cores=1
    )
    f = functools.partial(
        pl.kernel,
        mesh=mesh,
        compiler_params=pltpu.CompilerParams(needs_layout_passes=False),
        out_type=jax.ShapeDtypeStruct((LAYER,), jnp.float32),
        scratch_types=[
            pltpu.VMEM((N_NODES,), jnp.float32),
            pltpu.VMEM((N_COMPUTED * SLAB,), jnp.int32),
            pltpu.VMEM((N_COMPUTED * SLAB,), jnp.float32),
            pltpu.VMEM((NODES_PER_TILE,), jnp.float32),
            pltpu.VMEM_SHARED((2 * XCHG,), jnp.float32),
            pltpu.SemaphoreType.DMA,
            pltpu.SemaphoreType.DMA,
        ],
    )(_body)
    return f(src_all, w_all, inputs)


def kernel(inputs, edge_weight, edge_src, edge_dst):
    del edge_dst
    return _run_net(edge_src, edge_weight, inputs)


# speedup vs baseline: 1.0625x; 1.0240x over previous
import functools

import jax
import jax.numpy as jnp
from jax import lax
from jax.experimental import pallas as pl
from jax.experimental.pallas import tpu as pltpu
from jax.experimental.pallas import tpu_sc as plsc

N_INPUT = 1000
LAYER = 1000
N_LAYERS = 10
N_NODES = N_LAYERS * LAYER
IN_DEG = 16
N_COMPUTED = N_LAYERS - 1

LANES = 16
NUM_TILES = 16
NODES_PER_TILE = 64
GROUPS = NODES_PER_TILE // LANES
LAST_BASE = LAYER - NODES_PER_TILE
SLAB = NODES_PER_TILE * IN_DEG
XCHG = 1024


def _body(src_hbm, w_hbm, in_hbm, out_hbm,
          out_buf, my_src, my_w, pub, shared, sem0, sem1):
    c = lax.axis_index("c")
    t = lax.axis_index("s")

    @pl.when(c == 0)
    def _run():
        base = jnp.where(t == NUM_TILES - 1, LAST_BASE, t * NODES_PER_TILE)
        ebase = base * IN_DEG
        early = [
            pltpu.async_copy(src_hbm.at[pl.ds(ebase, SLAB)],
                             my_src.at[pl.ds(0, SLAB)], sem0),
            pltpu.async_copy(w_hbm.at[pl.ds(ebase, SLAB)],
                             my_w.at[pl.ds(0, SLAB)], sem0),
            pltpu.async_copy(in_hbm, out_buf.at[pl.ds(0, N_INPUT)], sem0),
        ]
        late = []
        for l in range(1, N_COMPUTED):
            off = ebase + l * LAYER * IN_DEG
            late.append(pltpu.async_copy(
                src_hbm.at[pl.ds(off, SLAB)],
                my_src.at[pl.ds(l * SLAB, SLAB)], sem1))
            late.append(pltpu.async_copy(
                w_hbm.at[pl.ds(off, SLAB)],
                my_w.at[pl.ds(l * SLAB, SLAB)], sem1))
        for cp in early:
            cp.wait()

        lane16 = jax.lax.iota(jnp.int32, 16) * IN_DEG

        def compute_layer(l):
            def gbody(g):
                cbase = lane16 + (l * SLAB + g * (LANES * IN_DEG))

                def gstep(j, acc):
                    idx = cbase + j
                    sv = plsc.load_gather(my_src, [idx])
                    wv = plsc.load_gather(my_w, [idx])
                    gv = plsc.load_gather(out_buf, [sv])
                    return acc + wv * gv

                acc = plsc.parallel_loop(
                    0, IN_DEG, unroll=16, carry=jnp.zeros((LANES,), jnp.float32)
                )(gstep)
                y = 1.0 / (1.0 + jnp.exp(-acc))
                pub[pl.ds(pl.multiple_of(g * LANES, LANES), LANES)] = y

            lax.fori_loop(0, GROUPS, lambda g, _: (gbody(g), 0)[1], 0)

        def exchange(l):
            sel = (l & 1) * XCHG
            pltpu.sync_copy(
                pub,
                shared.at[pl.ds(pl.multiple_of(sel + base, 8), NODES_PER_TILE)])
            plsc.subcore_barrier()
            pltpu.sync_copy(
                shared.at[pl.ds(pl.multiple_of(sel, 8), LAYER)],
                out_buf.at[pl.ds(pl.multiple_of((l + 1) * LAYER, 8), LAYER)],
            )

        compute_layer(0)
        for cp in late:
            cp.wait()
        exchange(0)

        def lstep(l, _):
            compute_layer(l)
            exchange(l)
            return 0
        lax.fori_loop(1, N_COMPUTED - 1, lstep, 0)

        compute_layer(N_COMPUTED - 1)
        pltpu.sync_copy(pub, out_hbm.at[pl.ds(base, NODES_PER_TILE)])


@jax.jit
def _run_net(src_all, w_all, inputs):
    mesh = plsc.VectorSubcoreMesh(
        core_axis_name="c", subcore_axis_name="s", num_cores=1
    )
    f = functools.partial(
        pl.kernel,
        mesh=mesh,
        compiler_params=pltpu.CompilerParams(needs_layout_passes=False),
        out_type=jax.ShapeDtypeStruct((LAYER,), jnp.float32),
        scratch_types=[
            pltpu.VMEM((N_NODES,), jnp.float32),
            pltpu.VMEM((N_COMPUTED * SLAB,), jnp.int32),
            pltpu.VMEM((N_COMPUTED * SLAB,), jnp.float32),
            pltpu.VMEM((NODES_PER_TILE,), jnp.float32),
            pltpu.VMEM_SHARED((2 * XCHG,), jnp.float32),
            pltpu.SemaphoreType.DMA,
            pltpu.SemaphoreType.DMA,
        ],
    )(_body)
    return f(src_all, w_all, inputs)


def kernel(inputs, edge_weight, edge_src, edge_dst):
    del edge_dst
    return _run_net(edge_src, edge_weight, inputs)
